# baseline (speedup 1.0000x reference)
import functools

import jax
import jax.numpy as jnp
from jax import lax
from jax.experimental import pallas as pl
from jax.experimental.pallas import tpu as pltpu
from jax.experimental.pallas import tpu_sc as plsc

_DIM = 128
_LEVELS = 16
_CLIP = 3.0 / (_DIM ** 0.5)
_STEP = (2.0 * _CLIP) / (_LEVELS - 1)
_INV_STEP = 1.0 / _STEP
_HALF = (_LEVELS - 1) / 2.0

_NC = 2
_NS = 16
_NW = _NC * _NS
_L = 16

_CHUNK = 256


def _compute_chunk(buf, table):
    tbl = table[...]

    @plsc.parallel_loop(0, _CHUNK, unroll=4)
    def _row(i):
        for j in range(_DIM // _L):
            v = buf[i, pl.ds(j * _L, _L)]
            vc = jnp.minimum(jnp.maximum(v, -_CLIP), _CLIP)
            t = vc * _INV_STEP + (_HALF + 0.5)
            idx = t.astype(jnp.int32)
            buf[i, pl.ds(j * _L, _L)] = tbl.at[idx].get(mode="promise_in_bounds")


def _sc_quant_body(y_hbm, o_hbm, b0, b1, table, is0, is1, os0, os1, *, row0=0):
    wid = lax.axis_index("s") * _NC + lax.axis_index("c")
    n = o_hbm.shape[0]
    rows_per_w = n // _NW
    chunks = rows_per_w // _CHUNK
    base_row = wid * rows_per_w
    bufs = (b0, b1)
    isems = (is0, is1)
    osems = (os0, os1)

    lvl = lax.iota(jnp.int32, _L).astype(jnp.float32)
    table[...] = (lvl - _HALF) * _STEP

    def in_slice(g):
        return y_hbm.at[pl.ds(row0 + base_row + g * _CHUNK, _CHUNK)]

    def out_slice(g):
        return o_hbm.at[pl.ds(base_row + g * _CHUNK, _CHUNK)]

    pltpu.async_copy(in_slice(0), bufs[0], isems[0])

    def outer(p, _):
        for b in range(2):
            g = p * 2 + b
            nb = 1 - b

            @pl.when(g >= 1)
            def _wait_prev_out():
                pltpu.make_async_copy(bufs[nb], out_slice(g - 1), osems[nb]).wait()

            @pl.when(g + 1 < chunks)
            def _start_next_in():
                pltpu.async_copy(in_slice(g + 1), bufs[nb], isems[nb])

            pltpu.make_async_copy(in_slice(g), bufs[b], isems[b]).wait()
            _compute_chunk(bufs[b], table)
            pltpu.async_copy(bufs[b], out_slice(g), osems[b])
        return _

    lax.fori_loop(0, chunks // 2, outer, 0)
    pltpu.make_async_copy(bufs[1], out_slice(chunks - 1), osems[1]).wait()


def _sc_part(y, n_sc, d, row0):
    return pl.kernel(
        functools.partial(_sc_quant_body, row0=row0),
        out_type=jax.ShapeDtypeStruct((n_sc, d), y.dtype),
        mesh=plsc.VectorSubcoreMesh(core_axis_name="c", subcore_axis_name="s"),
        scratch_types=[
            pltpu.VMEM((_CHUNK, _DIM), jnp.float32),
            pltpu.VMEM((_CHUNK, _DIM), jnp.float32),
            pltpu.VMEM((_L,), jnp.float32),
            pltpu.SemaphoreType.DMA,
            pltpu.SemaphoreType.DMA,
            pltpu.SemaphoreType.DMA,
            pltpu.SemaphoreType.DMA,
        ],
    )(y)


def _tc_quant_body(y_ref, o_ref):
    v = y_ref[...]
    vc = jnp.clip(v, -_CLIP, _CLIP)
    t = vc * _INV_STEP + _HALF
    k = jnp.floor(t + 0.5)
    o_ref[...] = (k - _HALF) * _STEP


_N_TC = 507904
_TC_BLK = 16384


def kernel(y):
    n, d = y.shape
    out_sc = _sc_part(y, n - _N_TC, d, _N_TC)
    out_tc_full = pl.pallas_call(
        _tc_quant_body,
        out_shape=jax.ShapeDtypeStruct((n, d), y.dtype),
        grid=(_N_TC // _TC_BLK,),
        in_specs=[pl.BlockSpec((_TC_BLK, d), lambda i: (i, 0))],
        out_specs=pl.BlockSpec((_TC_BLK, d), lambda i: (i, 0)),
    )(y)
    return lax.dynamic_update_slice(out_tc_full, out_sc, (_N_TC, 0))
